# Initial kernel scaffold; baseline (speedup 1.0000x reference)
import jax
import jax.numpy as jnp
from jax.experimental import pallas as pl


def kernel(x, edge_index, edge_weight, eps, W0, W1, W2):
    raise NotImplementedError("write your pallas kernel here")


# speedup vs baseline: 6.0592x; 6.0592x over previous
import functools

import jax
import jax.numpy as jnp
from jax import lax
from jax.experimental import pallas as pl
from jax.experimental.pallas import tpu as pltpu
from jax.experimental.pallas import tpu_sc as plsc

N_NODES = 10000
N_EDGES = 320000
D_FEAT = 128
D_HID = 64
D_EMB = 32

NC = 2
NS = 16
NW = NC * NS
EPW = N_EDGES // NW
CHUNK = 80
NCHUNK = EPW // CHUNK
STRIPE = 624
TAIL = N_NODES - NS * STRIPE


def _spmm_body(feat_hbm, src_hbm, dst_hbm, w_hbm, zeros_hbm, out_hbm,
               srcv, dstv, wv, rows, acc, sem):
    cid = lax.axis_index("c")
    sid = lax.axis_index("s")
    wid = sid * NC + cid

    soff = pl.multiple_of(sid * STRIPE, 8)
    pltpu.sync_copy(zeros_hbm.at[pl.ds(soff, STRIPE)],
                    acc.at[pl.ds(soff, STRIPE)])

    @pl.when(sid == 0)
    def _zero_tail():
        pltpu.sync_copy(zeros_hbm.at[pl.ds(NS * STRIPE, TAIL)],
                        acc.at[pl.ds(NS * STRIPE, TAIL)])

    plsc.subcore_barrier()

    e_base = pl.multiple_of(wid * EPW, 8)

    def chunk_body(j, carry):
        eb = pl.multiple_of(e_base + j * CHUNK, 8)
        pltpu.sync_copy(src_hbm.at[pl.ds(eb, CHUNK)], srcv)
        pltpu.sync_copy(dst_hbm.at[pl.ds(eb, CHUNK)], dstv)
        pltpu.sync_copy(w_hbm.at[pl.ds(eb, CHUNK)], wv)
        pltpu.async_copy(feat_hbm.at[srcv], rows, sem).wait()

        def e_body(eg, c2):
            w16 = wv[pl.ds(pl.multiple_of(eg * 16, 16), 16)]
            for de in range(16):
                e = eg * 16 + de
                wb = jnp.broadcast_to(w16[de], (16,))
                for q in range(D_HID // 16):
                    sl = pl.ds(q * 16, 16)
                    rows[e, sl] = rows[e, sl] * wb
            return c2

        lax.fori_loop(0, CHUNK // 16, e_body, 0)
        pltpu.sync_copy(rows, acc.at[dstv], add=True)
        return carry

    lax.fori_loop(0, NCHUNK, chunk_body, 0)
    plsc.subcore_barrier()
    pltpu.sync_copy(acc.at[pl.ds(soff, STRIPE)],
                    out_hbm.at[cid, pl.ds(soff, STRIPE)])

    @pl.when(sid == 0)
    def _out_tail():
        pltpu.sync_copy(acc.at[pl.ds(NS * STRIPE, TAIL)],
                        out_hbm.at[cid, pl.ds(NS * STRIPE, TAIL)])


_spmm_sc = pl.kernel(
    _spmm_body,
    out_type=jax.ShapeDtypeStruct((NC, N_NODES, D_HID), jnp.float32),
    mesh=plsc.VectorSubcoreMesh(core_axis_name="c", subcore_axis_name="s"),
    scratch_types=[
        pltpu.VMEM((CHUNK,), jnp.int32),
        pltpu.VMEM((CHUNK,), jnp.int32),
        pltpu.VMEM((CHUNK,), jnp.float32),
        pltpu.VMEM((CHUNK, D_HID), jnp.float32),
        pltpu.VMEM_SHARED((N_NODES, D_HID), jnp.float32),
        pltpu.SemaphoreType.DMA,
    ],
    compiler_params=pltpu.CompilerParams(use_tc_tiling_on_sc=False),
)


_BLK = 1000


def _mm1_body(x_ref, w_ref, o_ref):
    o_ref[...] = jnp.dot(x_ref[...], w_ref[...],
                         preferred_element_type=jnp.float32)


_mm1 = pl.pallas_call(
    _mm1_body,
    grid=(N_NODES // _BLK,),
    in_specs=[
        pl.BlockSpec((_BLK, D_FEAT), lambda i: (i, 0)),
        pl.BlockSpec((D_FEAT, D_HID), lambda i: (0, 0)),
    ],
    out_specs=pl.BlockSpec((_BLK, D_HID), lambda i: (i, 0)),
    out_shape=jax.ShapeDtypeStruct((N_NODES, D_HID), jnp.float32),
)


def _mid_body(p_ref, w_ref, o_ref):
    h = jnp.maximum(p_ref[0] + p_ref[1], 0.0)
    o_ref[...] = jnp.dot(h, w_ref[...], preferred_element_type=jnp.float32)


_mid = pl.pallas_call(
    _mid_body,
    grid=(N_NODES // _BLK,),
    in_specs=[
        pl.BlockSpec((NC, _BLK, D_HID), lambda i: (0, i, 0)),
        pl.BlockSpec((D_HID, 2 * D_EMB), lambda i: (0, 0)),
    ],
    out_specs=pl.BlockSpec((_BLK, 2 * D_EMB), lambda i: (i, 0)),
    out_shape=jax.ShapeDtypeStruct((N_NODES, 2 * D_EMB), jnp.float32),
)


def _fin_body(q_ref, eps_ref, o_ref):
    qs = q_ref[0] + q_ref[1]
    mu = qs[:, :D_EMB]
    ls2 = qs[:, D_EMB:]
    o_ref[...] = eps_ref[...] * jnp.exp(ls2 * 0.5) + mu


_fin = pl.pallas_call(
    _fin_body,
    grid=(N_NODES // _BLK,),
    in_specs=[
        pl.BlockSpec((NC, _BLK, 2 * D_EMB), lambda i: (0, i, 0)),
        pl.BlockSpec((_BLK, D_EMB), lambda i: (i, 0)),
    ],
    out_specs=pl.BlockSpec((_BLK, D_EMB), lambda i: (i, 0)),
    out_shape=jax.ShapeDtypeStruct((N_NODES, D_EMB), jnp.float32),
)


def kernel(x, edge_index, edge_weight, eps, W0, W1, W2):
    src = edge_index[0].astype(jnp.int32)
    dst = edge_index[1].astype(jnp.int32)
    w = edge_weight.astype(jnp.float32)
    zeros = jnp.zeros((N_NODES, D_HID), jnp.float32)

    xw = _mm1(x, W0)
    p = _spmm_sc(xw, src, dst, w, zeros)
    Wc = jnp.concatenate([W1, W2], axis=1)
    hw = _mid(p, Wc)
    q = _spmm_sc(hw, src, dst, w, zeros)
    z = _fin(q, eps)
    return z


# speedup vs baseline: 11.3893x; 1.8797x over previous
import functools

import jax
import jax.numpy as jnp
from jax import lax
from jax.experimental import pallas as pl
from jax.experimental.pallas import tpu as pltpu
from jax.experimental.pallas import tpu_sc as plsc

N_NODES = 10000
N_EDGES = 320000
D_FEAT = 128
D_HID = 64
D_EMB = 32

NC = 2
NS = 16
NW = NC * NS
EPW = N_EDGES // NW
CHUNK = 400
NCHUNK = EPW // CHUNK
STRIPE = 624
TAIL = N_NODES - NS * STRIPE


def _spmm_body(feat_hbm, src_hbm, dst_hbm, w_hbm, zeros_hbm, out_hbm,
               srcv, dstv, wv, rows, acc, sem):
    cid = lax.axis_index("c")
    sid = lax.axis_index("s")
    wid = sid * NC + cid

    soff = pl.multiple_of(sid * STRIPE, 8)
    pltpu.sync_copy(zeros_hbm.at[pl.ds(soff, STRIPE)],
                    acc.at[pl.ds(soff, STRIPE)])

    @pl.when(sid == 0)
    def _zero_tail():
        pltpu.sync_copy(zeros_hbm.at[pl.ds(NS * STRIPE, TAIL)],
                        acc.at[pl.ds(NS * STRIPE, TAIL)])

    plsc.subcore_barrier()

    e_base = pl.multiple_of(wid * EPW, 8)

    def chunk_body(j, carry):
        eb = pl.multiple_of(e_base + j * CHUNK, 8)
        pltpu.sync_copy(src_hbm.at[pl.ds(eb, CHUNK)], srcv)
        pltpu.sync_copy(dst_hbm.at[pl.ds(eb, CHUNK)], dstv)
        pltpu.sync_copy(w_hbm.at[pl.ds(eb, CHUNK)], wv)
        pltpu.async_copy(feat_hbm.at[srcv], rows, sem).wait()

        def e_body(eg, c2):
            w16 = wv[pl.ds(pl.multiple_of(eg * 16, 16), 16)]
            for de in range(16):
                e = eg * 16 + de
                wb = jnp.broadcast_to(w16[de], (16,))
                for q in range(D_HID // 16):
                    sl = pl.ds(q * 16, 16)
                    rows[e, sl] = rows[e, sl] * wb
            return c2

        lax.fori_loop(0, CHUNK // 16, e_body, 0)
        pltpu.sync_copy(rows, acc.at[dstv], add=True)
        return carry

    lax.fori_loop(0, NCHUNK, chunk_body, 0)
    plsc.subcore_barrier()
    pltpu.sync_copy(acc.at[pl.ds(soff, STRIPE)],
                    out_hbm.at[cid, pl.ds(soff, STRIPE)])

    @pl.when(sid == 0)
    def _out_tail():
        pltpu.sync_copy(acc.at[pl.ds(NS * STRIPE, TAIL)],
                        out_hbm.at[cid, pl.ds(NS * STRIPE, TAIL)])


_spmm_sc = pl.kernel(
    _spmm_body,
    out_type=jax.ShapeDtypeStruct((NC, N_NODES, D_HID), jnp.float32),
    mesh=plsc.VectorSubcoreMesh(core_axis_name="c", subcore_axis_name="s"),
    scratch_types=[
        pltpu.VMEM((CHUNK,), jnp.int32),
        pltpu.VMEM((CHUNK,), jnp.int32),
        pltpu.VMEM((CHUNK,), jnp.float32),
        pltpu.VMEM((CHUNK, D_HID), jnp.float32),
        pltpu.VMEM_SHARED((N_NODES, D_HID), jnp.float32),
        pltpu.SemaphoreType.DMA,
    ],
    compiler_params=pltpu.CompilerParams(use_tc_tiling_on_sc=False),
)


_BLK = 1000


def _mm1_body(x_ref, w_ref, o_ref):
    o_ref[...] = jnp.dot(x_ref[...], w_ref[...],
                         preferred_element_type=jnp.float32)


_mm1 = pl.pallas_call(
    _mm1_body,
    grid=(N_NODES // _BLK,),
    in_specs=[
        pl.BlockSpec((_BLK, D_FEAT), lambda i: (i, 0)),
        pl.BlockSpec((D_FEAT, D_HID), lambda i: (0, 0)),
    ],
    out_specs=pl.BlockSpec((_BLK, D_HID), lambda i: (i, 0)),
    out_shape=jax.ShapeDtypeStruct((N_NODES, D_HID), jnp.float32),
)


def _mid_body(p_ref, w_ref, o_ref):
    h = jnp.maximum(p_ref[0] + p_ref[1], 0.0)
    o_ref[...] = jnp.dot(h, w_ref[...], preferred_element_type=jnp.float32)


_mid = pl.pallas_call(
    _mid_body,
    grid=(N_NODES // _BLK,),
    in_specs=[
        pl.BlockSpec((NC, _BLK, D_HID), lambda i: (0, i, 0)),
        pl.BlockSpec((D_HID, 2 * D_EMB), lambda i: (0, 0)),
    ],
    out_specs=pl.BlockSpec((_BLK, 2 * D_EMB), lambda i: (i, 0)),
    out_shape=jax.ShapeDtypeStruct((N_NODES, 2 * D_EMB), jnp.float32),
)


def _fin_body(q_ref, eps_ref, o_ref):
    qs = q_ref[0] + q_ref[1]
    mu = qs[:, :D_EMB]
    ls2 = qs[:, D_EMB:]
    o_ref[...] = eps_ref[...] * jnp.exp(ls2 * 0.5) + mu


_fin = pl.pallas_call(
    _fin_body,
    grid=(N_NODES // _BLK,),
    in_specs=[
        pl.BlockSpec((NC, _BLK, 2 * D_EMB), lambda i: (0, i, 0)),
        pl.BlockSpec((_BLK, D_EMB), lambda i: (i, 0)),
    ],
    out_specs=pl.BlockSpec((_BLK, D_EMB), lambda i: (i, 0)),
    out_shape=jax.ShapeDtypeStruct((N_NODES, D_EMB), jnp.float32),
)


def kernel(x, edge_index, edge_weight, eps, W0, W1, W2):
    src = edge_index[0].astype(jnp.int32)
    dst = edge_index[1].astype(jnp.int32)
    w = edge_weight.astype(jnp.float32)
    zeros = jnp.zeros((N_NODES, D_HID), jnp.float32)

    xw = _mm1(x, W0)
    p = _spmm_sc(xw, src, dst, w, zeros)
    Wc = jnp.concatenate([W1, W2], axis=1)
    hw = _mid(p, Wc)
    q = _spmm_sc(hw, src, dst, w, zeros)
    z = _fin(q, eps)
    return z


# speedup vs baseline: 13.1205x; 1.1520x over previous
import functools

import jax
import jax.numpy as jnp
from jax import lax
from jax.experimental import pallas as pl
from jax.experimental.pallas import tpu as pltpu
from jax.experimental.pallas import tpu_sc as plsc

N_NODES = 10000
N_EDGES = 320000
D_FEAT = 128
D_HID = 64
D_EMB = 32

NC = 2
NS = 16
NW = NC * NS
EPW = N_EDGES // NW
CHUNK = 1000
NCHUNK = EPW // CHUNK
STRIPE = 624
TAIL = N_NODES - NS * STRIPE


def _spmm_body(feat_hbm, src_hbm, dst_hbm, w_hbm, zeros_hbm, out_hbm,
               srcv, dstv, wv, rows, acc, sem):
    cid = lax.axis_index("c")
    sid = lax.axis_index("s")
    wid = sid * NC + cid

    soff = pl.multiple_of(sid * STRIPE, 8)
    pltpu.sync_copy(zeros_hbm.at[pl.ds(soff, STRIPE)],
                    acc.at[pl.ds(soff, STRIPE)])

    @pl.when(sid == 0)
    def _zero_tail():
        pltpu.sync_copy(zeros_hbm.at[pl.ds(NS * STRIPE, TAIL)],
                        acc.at[pl.ds(NS * STRIPE, TAIL)])

    plsc.subcore_barrier()

    e_base = pl.multiple_of(wid * EPW, 8)

    def chunk_body(j, carry):
        eb = pl.multiple_of(e_base + j * CHUNK, 8)
        pltpu.sync_copy(src_hbm.at[pl.ds(eb, CHUNK)], srcv)
        pltpu.sync_copy(dst_hbm.at[pl.ds(eb, CHUNK)], dstv)
        pltpu.sync_copy(w_hbm.at[pl.ds(eb, CHUNK)], wv)
        pltpu.async_copy(feat_hbm.at[srcv], rows, sem).wait()

        def e_body(eg, c2):
            w16 = wv[pl.ds(pl.multiple_of(eg * 16, 16), 16)]
            for de in range(16):
                e = eg * 16 + de
                wb = jnp.broadcast_to(w16[de], (16,))
                for q in range(D_HID // 16):
                    sl = pl.ds(q * 16, 16)
                    rows[e, sl] = rows[e, sl] * wb
            return c2

        lax.fori_loop(0, CHUNK // 16, e_body, 0)
        pltpu.sync_copy(rows, acc.at[dstv], add=True)
        return carry

    lax.fori_loop(0, NCHUNK, chunk_body, 0)
    plsc.subcore_barrier()
    pltpu.sync_copy(acc.at[pl.ds(soff, STRIPE)],
                    out_hbm.at[cid, pl.ds(soff, STRIPE)])

    @pl.when(sid == 0)
    def _out_tail():
        pltpu.sync_copy(acc.at[pl.ds(NS * STRIPE, TAIL)],
                        out_hbm.at[cid, pl.ds(NS * STRIPE, TAIL)])


_spmm_sc = pl.kernel(
    _spmm_body,
    out_type=jax.ShapeDtypeStruct((NC, N_NODES, D_HID), jnp.float32),
    mesh=plsc.VectorSubcoreMesh(core_axis_name="c", subcore_axis_name="s"),
    scratch_types=[
        pltpu.VMEM((CHUNK,), jnp.int32),
        pltpu.VMEM((CHUNK,), jnp.int32),
        pltpu.VMEM((CHUNK,), jnp.float32),
        pltpu.VMEM((CHUNK, D_HID), jnp.float32),
        pltpu.VMEM_SHARED((N_NODES, D_HID), jnp.float32),
        pltpu.SemaphoreType.DMA,
    ],
    compiler_params=pltpu.CompilerParams(use_tc_tiling_on_sc=False),
)


_BLK = 1000


def _mm1_body(x_ref, w_ref, o_ref):
    o_ref[...] = jnp.dot(x_ref[...], w_ref[...],
                         preferred_element_type=jnp.float32)


_mm1 = pl.pallas_call(
    _mm1_body,
    grid=(N_NODES // _BLK,),
    in_specs=[
        pl.BlockSpec((_BLK, D_FEAT), lambda i: (i, 0)),
        pl.BlockSpec((D_FEAT, D_HID), lambda i: (0, 0)),
    ],
    out_specs=pl.BlockSpec((_BLK, D_HID), lambda i: (i, 0)),
    out_shape=jax.ShapeDtypeStruct((N_NODES, D_HID), jnp.float32),
)


def _mid_body(p_ref, w_ref, o_ref):
    h = jnp.maximum(p_ref[0] + p_ref[1], 0.0)
    o_ref[...] = jnp.dot(h, w_ref[...], preferred_element_type=jnp.float32)


_mid = pl.pallas_call(
    _mid_body,
    grid=(N_NODES // _BLK,),
    in_specs=[
        pl.BlockSpec((NC, _BLK, D_HID), lambda i: (0, i, 0)),
        pl.BlockSpec((D_HID, 2 * D_EMB), lambda i: (0, 0)),
    ],
    out_specs=pl.BlockSpec((_BLK, 2 * D_EMB), lambda i: (i, 0)),
    out_shape=jax.ShapeDtypeStruct((N_NODES, 2 * D_EMB), jnp.float32),
)


def _fin_body(q_ref, eps_ref, o_ref):
    qs = q_ref[0] + q_ref[1]
    mu = qs[:, :D_EMB]
    ls2 = qs[:, D_EMB:]
    o_ref[...] = eps_ref[...] * jnp.exp(ls2 * 0.5) + mu


_fin = pl.pallas_call(
    _fin_body,
    grid=(N_NODES // _BLK,),
    in_specs=[
        pl.BlockSpec((NC, _BLK, 2 * D_EMB), lambda i: (0, i, 0)),
        pl.BlockSpec((_BLK, D_EMB), lambda i: (i, 0)),
    ],
    out_specs=pl.BlockSpec((_BLK, D_EMB), lambda i: (i, 0)),
    out_shape=jax.ShapeDtypeStruct((N_NODES, D_EMB), jnp.float32),
)


def kernel(x, edge_index, edge_weight, eps, W0, W1, W2):
    src = edge_index[0].astype(jnp.int32)
    dst = edge_index[1].astype(jnp.int32)
    w = edge_weight.astype(jnp.float32)
    zeros = jnp.zeros((N_NODES, D_HID), jnp.float32)

    xw = _mm1(x, W0)
    p = _spmm_sc(xw, src, dst, w, zeros)
    Wc = jnp.concatenate([W1, W2], axis=1)
    hw = _mid(p, Wc)
    q = _spmm_sc(hw, src, dst, w, zeros)
    z = _fin(q, eps)
    return z
